# baseline (speedup 1.0000x reference)
import functools

import jax
import jax.numpy as jnp
from jax import lax
from jax.experimental import pallas as pl
from jax.experimental.pallas import tpu as pltpu
from jax.experimental.pallas import tpu_sc as plsc

N = 10000
E = 160000
F_IN = 256
H = 512
C = 40
T = 8

ROW_BLK = 1000
NCHUNK = 4
W = 128
NP2 = N + 8
NPA = 10240
HALF = 5120
ACC_R = HALF + 8
KB = 64
NBC = 168
CAPT = NBC * KB
CAP = 16 * CAPT
RPT = HALF // 16


def _gelu(x):
    return 0.5 * x * (1.0 + lax.erf(x * (2.0 ** -0.5)))


def _proj_body(x_ref, w_ref, b_ref, o_ref):
    o_ref[...] = (
        jnp.dot(x_ref[...], w_ref[...], preferred_element_type=jnp.float32)
        + b_ref[...]
    )


def _proj(x, w, b):
    m, k = x.shape
    n = w.shape[1]
    return pl.pallas_call(
        _proj_body,
        grid=(m // ROW_BLK,),
        in_specs=[
            pl.BlockSpec((ROW_BLK, k), lambda i: (i, 0)),
            pl.BlockSpec((k, n), lambda i: (0, 0)),
            pl.BlockSpec((1, n), lambda i: (0, 0)),
        ],
        out_specs=pl.BlockSpec((ROW_BLK, n), lambda i: (i, 0)),
        out_shape=jax.ShapeDtypeStruct((m, n), jnp.float32),
    )(x, w, b.reshape(1, n))


def _ginit_body(h_ref, a_ref, gp_ref, gm_ref):
    g = a_ref[...] * h_ref[...]
    gp_ref[0] = g
    gm_ref[0] = -g


def _ginit(h, a2d):
    return pl.pallas_call(
        _ginit_body,
        grid=(NCHUNK, N // ROW_BLK),
        in_specs=[
            pl.BlockSpec((ROW_BLK, W), lambda c, i: (i, c)),
            pl.BlockSpec((ROW_BLK, 1), lambda c, i: (i, 0)),
        ],
        out_specs=[
            pl.BlockSpec((1, ROW_BLK, W), lambda c, i: (c, i, 0)),
            pl.BlockSpec((1, ROW_BLK, W), lambda c, i: (c, i, 0)),
        ],
        out_shape=[
            jax.ShapeDtypeStruct((NCHUNK, NP2, W), jnp.float32),
            jax.ShapeDtypeStruct((NCHUNK, NP2, W), jnp.float32),
        ],
    )(h, a2d)


def _tupd_body(t_inv, u_ref, a_ref, prev_ref, o_ref, gp_ref, gm_ref):
    tn = a_ref[...] * u_ref[0] * t_inv
    o_ref[...] = prev_ref[...] + tn
    g = a_ref[...] * tn
    gp_ref[0] = g
    gm_ref[0] = -g


def _term_update(u4, a2d, prev, t):
    return pl.pallas_call(
        functools.partial(_tupd_body, 1.0 / t),
        grid=(NCHUNK, N // ROW_BLK),
        in_specs=[
            pl.BlockSpec((1, ROW_BLK, W), lambda c, i: (c, i, 0)),
            pl.BlockSpec((ROW_BLK, 1), lambda c, i: (i, 0)),
            pl.BlockSpec((ROW_BLK, W), lambda c, i: (i, c)),
        ],
        out_specs=[
            pl.BlockSpec((ROW_BLK, W), lambda c, i: (i, c)),
            pl.BlockSpec((1, ROW_BLK, W), lambda c, i: (c, i, 0)),
            pl.BlockSpec((1, ROW_BLK, W), lambda c, i: (c, i, 0)),
        ],
        out_shape=[
            jax.ShapeDtypeStruct((N, H), jnp.float32),
            jax.ShapeDtypeStruct((NCHUNK, NP2, W), jnp.float32),
            jax.ShapeDtypeStruct((NCHUNK, NP2, W), jnp.float32),
        ],
    )(u4, a2d, prev)


def _tfin_body(t_inv, has_res, u_ref, a_ref, prev_ref, x_ref, o_ref):
    tn = a_ref[...] * u_ref[0] * t_inv
    o = prev_ref[...] + tn
    if has_res:
        o = o + x_ref[...]
    o_ref[...] = o


def _term_final(u4, a2d, prev, xres, t):
    return pl.pallas_call(
        functools.partial(_tfin_body, 1.0 / t, xres is not None),
        grid=(NCHUNK, N // ROW_BLK),
        in_specs=[
            pl.BlockSpec((1, ROW_BLK, W), lambda c, i: (c, i, 0)),
            pl.BlockSpec((ROW_BLK, 1), lambda c, i: (i, 0)),
            pl.BlockSpec((ROW_BLK, W), lambda c, i: (i, c)),
            pl.BlockSpec((ROW_BLK, W), lambda c, i: (i, c)),
        ],
        out_specs=pl.BlockSpec((ROW_BLK, W), lambda c, i: (i, c)),
        out_shape=jax.ShapeDtypeStruct((N, H), jnp.float32),
    )(u4, a2d, prev, xres if xres is not None else prev)


def _spmm_body(gp_hbm, gm_hbm, colsF, colsB, rowsF, rowsB, nbp_hbm, zeros_hbm,
               u4_hbm, colbuf, rowbuf, nbbuf, b0, b1, b2, b3, zbuf, acc,
               gsem, ssem):
    cc = lax.axis_index("c")
    s = lax.axis_index("s")
    bufs = (b0, b1, b2, b3)
    pltpu.sync_copy(zeros_hbm, zbuf)
    pltpu.sync_copy(nbp_hbm.at[cc, s], nbbuf)
    for chunk in range(NCHUNK):
        for z in range(5):
            pltpu.sync_copy(zbuf, acc.at[pl.ds(s * RPT + z * 64, 64)])
        plsc.subcore_barrier()
        for dirn in range(2):
            cols = colsF if dirn == 0 else colsB
            rows_g = rowsF if dirn == 0 else rowsB
            gsrc = gp_hbm if dirn == 0 else gm_hbm
            pltpu.sync_copy(cols.at[chunk, cc, s], colbuf)
            pltpu.sync_copy(rows_g.at[cc, s], rowbuf)
            nquads = nbbuf[...][dirn]

            def body(q, carry):
                i = 4 * q
                for j in range(4):
                    @pl.when(q > 0)
                    def _(j=j):
                        pltpu.make_async_copy(gsrc.at[pl.ds(0, KB)], bufs[j],
                                              ssem).wait()
                for j in range(4):
                    pltpu.async_copy(gsrc.at[colbuf.at[i + j]], bufs[j], gsem)
                for j in range(4):
                    pltpu.make_async_copy(gsrc.at[pl.ds(0, KB)], bufs[j],
                                          gsem).wait()
                for j in range(4):
                    pltpu.async_copy(bufs[j], acc.at[rowbuf.at[i + j]], ssem,
                                     add=True)
                return carry

            lax.fori_loop(0, nquads, body, 0)
            for j in range(4):
                pltpu.make_async_copy(gsrc.at[pl.ds(0, KB)], bufs[j],
                                      ssem).wait()
        plsc.subcore_barrier()
        pltpu.sync_copy(acc.at[pl.ds(s * RPT, RPT)],
                        u4_hbm.at[chunk, pl.ds(cc * HALF + s * RPT, RPT)])
        plsc.subcore_barrier()


@functools.cache
def _spmm_kernel():
    return pl.kernel(
        _spmm_body,
        mesh=plsc.VectorSubcoreMesh(core_axis_name="c", subcore_axis_name="s"),
        out_type=jax.ShapeDtypeStruct((NCHUNK, NPA, W), jnp.float32),
        scratch_types=[
            pltpu.VMEM((NBC, KB), jnp.int32),
            pltpu.VMEM((NBC, KB), jnp.int32),
            pltpu.VMEM((16,), jnp.int32),
            pltpu.VMEM((KB, W), jnp.float32),
            pltpu.VMEM((KB, W), jnp.float32),
            pltpu.VMEM((KB, W), jnp.float32),
            pltpu.VMEM((KB, W), jnp.float32),
            pltpu.VMEM((64, W), jnp.float32),
            pltpu.VMEM_SHARED((ACC_R, W), jnp.float32),
            pltpu.SemaphoreType.DMA,
            pltpu.SemaphoreType.DMA,
        ],
    )


def _spmm_call(*args):
    return _spmm_kernel()(*args)


def _mlp_body(x_ref, w1_ref, b1_ref, w2_ref, b2_ref, o_ref):
    x = x_ref[...]
    h = _gelu(jnp.dot(x, w1_ref[...], preferred_element_type=jnp.float32) + b1_ref[...])
    xm = jnp.dot(h, w2_ref[...], preferred_element_type=jnp.float32) + b2_ref[...]
    xs = xm + x
    nrm = jnp.sqrt(jnp.sum(xs * xs, axis=1, keepdims=True))
    xn = xs / jnp.maximum(nrm, 1e-12)
    o_ref[...] = xn + xm


def _mlp_norm(x, w1, b1, w2, b2):
    m = x.shape[0]
    return pl.pallas_call(
        _mlp_body,
        grid=(m // ROW_BLK,),
        in_specs=[
            pl.BlockSpec((ROW_BLK, H), lambda i: (i, 0)),
            pl.BlockSpec((H, H), lambda i: (0, 0)),
            pl.BlockSpec((1, H), lambda i: (0, 0)),
            pl.BlockSpec((H, H), lambda i: (0, 0)),
            pl.BlockSpec((1, H), lambda i: (0, 0)),
        ],
        out_specs=pl.BlockSpec((ROW_BLK, H), lambda i: (i, 0)),
        out_shape=jax.ShapeDtypeStruct((m, H), jnp.float32),
    )(x, w1, b1.reshape(1, H), w2, b2.reshape(1, H))


CPAD = 128


def _head_body(x_ref, w1_ref, b1_ref, w2_ref, b2_ref, w3_ref, b3_ref,
               w4_ref, b4_ref, o_ref):
    h = _gelu(jnp.dot(x_ref[...], w1_ref[...], preferred_element_type=jnp.float32) + b1_ref[...])
    h = _gelu(jnp.dot(h, w2_ref[...], preferred_element_type=jnp.float32) + b2_ref[...])
    h = _gelu(jnp.dot(h, w3_ref[...], preferred_element_type=jnp.float32) + b3_ref[...])
    logits = jnp.dot(h, w4_ref[...], preferred_element_type=jnp.float32) + b4_ref[...]
    mx = jnp.max(logits, axis=1, keepdims=True)
    z = logits - mx
    lse = jnp.log(jnp.sum(jnp.exp(z), axis=1, keepdims=True))
    o_ref[...] = z - lse


def _head(x, w1, b1, w2, b2, w3, b3, w4, b4):
    m = x.shape[0]
    w4p = jnp.zeros((H, CPAD), jnp.float32).at[:, :C].set(w4)
    b4p = jnp.full((1, CPAD), -1e30, jnp.float32).at[0, :C].set(b4)
    out = pl.pallas_call(
        _head_body,
        grid=(m // ROW_BLK,),
        in_specs=[
            pl.BlockSpec((ROW_BLK, H), lambda i: (i, 0)),
            pl.BlockSpec((H, H), lambda i: (0, 0)),
            pl.BlockSpec((1, H), lambda i: (0, 0)),
            pl.BlockSpec((H, H), lambda i: (0, 0)),
            pl.BlockSpec((1, H), lambda i: (0, 0)),
            pl.BlockSpec((H, H), lambda i: (0, 0)),
            pl.BlockSpec((1, H), lambda i: (0, 0)),
            pl.BlockSpec((H, CPAD), lambda i: (0, 0)),
            pl.BlockSpec((1, CPAD), lambda i: (0, 0)),
        ],
        out_specs=pl.BlockSpec((ROW_BLK, CPAD), lambda i: (i, 0)),
        out_shape=jax.ShapeDtypeStruct((m, CPAD), jnp.float32),
    )(x, w1, b1.reshape(1, H), w2, b2.reshape(1, H), w3, b3.reshape(1, H),
      w4p, b4p)
    return out[:, :C]


def _edge_slabs(src, dst):
    def part(col, row):
        half = (row >= HALF).astype(jnp.int32)
        local = row - half * HALF
        m0 = 1 - half
        q = jnp.where(m0, jnp.cumsum(m0) - 1, jnp.cumsum(half) - 1)
        pos = half * CAP + (q % 16) * CAPT + q // 16
        cols_arr = jnp.zeros((2 * CAP,), jnp.int32).at[pos].set(col)
        rows_arr = jnp.full((2 * CAP,), HALF, jnp.int32).at[pos].set(local)
        cnt0 = jnp.sum(m0)
        cnt = jnp.stack([cnt0, E - cnt0])
        t = jnp.arange(16, dtype=jnp.int32)
        per_tile = jnp.maximum(cnt[:, None] - t[None, :] + 15, 0) // 16
        nbatch = (per_tile + KB - 1) // KB
        npair = jnp.maximum((nbatch + 3) // 4, 1)
        return (cols_arr.reshape(2, 16, NBC, KB),
                rows_arr.reshape(2, 16, NBC, KB),
                npair.astype(jnp.int32))

    colF, rowsF, npF = part(src, dst)
    colB, rowsB, npB = part(dst, src)
    nbp = jnp.zeros((2, 16, 16), jnp.int32)
    nbp = nbp.at[:, :, 0].set(npF).at[:, :, 1].set(npB)
    offs = (jnp.arange(NCHUNK, dtype=jnp.int32) * NP2).reshape(NCHUNK, 1, 1, 1, 1)
    colsF4 = colF[None] + offs
    colsB4 = colB[None] + offs
    return colsF4, colsB4, rowsF, rowsB, nbp


def _conv(x, w, b, residual, a2d, cols_rows, zeros128):
    colsF4, colsB4, rowsF, rowsB, nbp = cols_rows
    h = _proj(x, w, b)
    gp, gm = _ginit(h, a2d)
    out = h
    for t in range(1, T + 1):
        u4 = _spmm_call(gp.reshape(NCHUNK * NP2, W), gm.reshape(NCHUNK * NP2, W),
                        colsF4, colsB4, rowsF, rowsB, nbp, zeros128)
        if t < T:
            out, gp, gm = _term_update(u4, a2d, out, t)
        else:
            out = _term_final(u4, a2d, out, x if residual else None, t)
    return out


def kernel(x_in, edge_index, convW0, convb0, convW, convb, mlpW1, mlpb1,
           mlpW2, mlpb2, finW1, finb1, finW2, finb2, finW3, finb3,
           finW4, finb4):
    src = edge_index[0]
    dst = edge_index[1]
    deg = jnp.zeros((N,), jnp.float32).at[src].add(1.0).at[dst].add(1.0)
    a2d = (1.0 / jnp.sqrt(deg + 1.0)).reshape(N, 1)
    cols_rows = _edge_slabs(src, dst)
    zeros128 = jnp.zeros((64, W), jnp.float32)

    x = x_in
    for i in range(3):
        if i == 0:
            x = _conv(x, convW0, convb0, False, a2d, cols_rows, zeros128)
        else:
            x = _conv(x, convW[i - 1], convb[i - 1], True, a2d, cols_rows,
                      zeros128)
        x = _mlp_norm(x, mlpW1[i], mlpb1[i], mlpW2[i], mlpb2[i])
    return _head(x, finW1, finb1, finW2, finb2, finW3, finb3, finW4, finb4)


# speedup vs baseline: 1.2378x; 1.2378x over previous
import functools

import jax
import jax.numpy as jnp
from jax import lax
from jax.experimental import pallas as pl
from jax.experimental.pallas import tpu as pltpu
from jax.experimental.pallas import tpu_sc as plsc

N = 10000
E = 160000
F_IN = 256
H = 512
C = 40
T = 8

ROW_BLK = 1000
NP2 = N + 8
NPA = 10240
OROWS = 1280
ACC_R = OROWS + 8
KB = 32
BE = 4 * KB
NBC = 48
CAPT = NBC * KB
ACC4 = ACC_R * 4
RPT = OROWS * 4 // 16


def _gelu(x):
    return 0.5 * x * (1.0 + lax.erf(x * (2.0 ** -0.5)))


def _proj_body(x_ref, w_ref, b_ref, o_ref):
    o_ref[...] = (
        jnp.dot(x_ref[...], w_ref[...], preferred_element_type=jnp.float32)
        + b_ref[...]
    )


def _proj(x, w, b):
    m, k = x.shape
    n = w.shape[1]
    return pl.pallas_call(
        _proj_body,
        grid=(m // ROW_BLK,),
        in_specs=[
            pl.BlockSpec((ROW_BLK, k), lambda i: (i, 0)),
            pl.BlockSpec((k, n), lambda i: (0, 0)),
            pl.BlockSpec((1, n), lambda i: (0, 0)),
        ],
        out_specs=pl.BlockSpec((ROW_BLK, n), lambda i: (i, 0)),
        out_shape=jax.ShapeDtypeStruct((m, n), jnp.float32),
    )(x, w, b.reshape(1, n))


def _ginit_body(h_ref, a_ref, gp_ref, gm_ref):
    g = a_ref[...] * h_ref[...]
    gp_ref[...] = g
    gm_ref[...] = -g


def _ginit(h, a2d):
    return pl.pallas_call(
        _ginit_body,
        grid=(N // ROW_BLK,),
        in_specs=[
            pl.BlockSpec((ROW_BLK, H), lambda i: (i, 0)),
            pl.BlockSpec((ROW_BLK, 1), lambda i: (i, 0)),
        ],
        out_specs=[
            pl.BlockSpec((ROW_BLK, H), lambda i: (i, 0)),
            pl.BlockSpec((ROW_BLK, H), lambda i: (i, 0)),
        ],
        out_shape=[
            jax.ShapeDtypeStruct((NP2, H), jnp.float32),
            jax.ShapeDtypeStruct((NP2, H), jnp.float32),
        ],
    )(h, a2d)


def _tupd_body(t_inv, u_ref, a_ref, prev_ref, o_ref, gp_ref, gm_ref):
    tn = a_ref[...] * u_ref[...] * t_inv
    o_ref[...] = prev_ref[...] + tn
    g = a_ref[...] * tn
    gp_ref[...] = g
    gm_ref[...] = -g


def _term_update(u, a2d, prev, t):
    return pl.pallas_call(
        functools.partial(_tupd_body, 1.0 / t),
        grid=(N // ROW_BLK,),
        in_specs=[
            pl.BlockSpec((ROW_BLK, H), lambda i: (i, 0)),
            pl.BlockSpec((ROW_BLK, 1), lambda i: (i, 0)),
            pl.BlockSpec((ROW_BLK, H), lambda i: (i, 0)),
        ],
        out_specs=[
            pl.BlockSpec((ROW_BLK, H), lambda i: (i, 0)),
            pl.BlockSpec((ROW_BLK, H), lambda i: (i, 0)),
            pl.BlockSpec((ROW_BLK, H), lambda i: (i, 0)),
        ],
        out_shape=[
            jax.ShapeDtypeStruct((N, H), jnp.float32),
            jax.ShapeDtypeStruct((NP2, H), jnp.float32),
            jax.ShapeDtypeStruct((NP2, H), jnp.float32),
        ],
    )(u, a2d, prev)


def _tfin_body(t_inv, has_res, u_ref, a_ref, prev_ref, x_ref, o_ref):
    tn = a_ref[...] * u_ref[...] * t_inv
    o = prev_ref[...] + tn
    if has_res:
        o = o + x_ref[...]
    o_ref[...] = o


def _term_final(u, a2d, prev, xres, t):
    return pl.pallas_call(
        functools.partial(_tfin_body, 1.0 / t, xres is not None),
        grid=(N // ROW_BLK,),
        in_specs=[
            pl.BlockSpec((ROW_BLK, H), lambda i: (i, 0)),
            pl.BlockSpec((ROW_BLK, 1), lambda i: (i, 0)),
            pl.BlockSpec((ROW_BLK, H), lambda i: (i, 0)),
            pl.BlockSpec((ROW_BLK, H), lambda i: (i, 0)),
        ],
        out_specs=pl.BlockSpec((ROW_BLK, H), lambda i: (i, 0)),
        out_shape=jax.ShapeDtypeStruct((N, H), jnp.float32),
    )(u, a2d, prev, xres if xres is not None else prev)


def _spmm_body(gp_hbm, gm_hbm, colsF, colsB, rowsF, rowsB, nbp_hbm, zeros_hbm,
               u_hbm, colbuf, rowbuf, nbbuf, buf0, buf1, acc,
               sem0, sem1):
    cc = lax.axis_index("c")
    s = lax.axis_index("s")
    pltpu.sync_copy(nbp_hbm.at[cc, s], nbbuf)
    for o in range(4):
        oct_ = cc * 4 + o
        for z in range(5):
            pltpu.sync_copy(zeros_hbm, acc.at[pl.ds(s * RPT + z * 64, 64)])
        plsc.subcore_barrier()
        for dirn in range(2):
            cols = colsF if dirn == 0 else colsB
            rows_g = rowsF if dirn == 0 else rowsB
            gsrc = gp_hbm if dirn == 0 else gm_hbm
            pltpu.sync_copy(cols.at[oct_, s], colbuf)
            pltpu.sync_copy(rows_g.at[oct_, s], rowbuf)
            npairs = nbbuf[...][dirn * 4 + o]
            pltpu.async_copy(gsrc.at[colbuf.at[0]], buf0, sem0)

            def body(k, carry):
                i = 2 * k
                pltpu.async_copy(gsrc.at[colbuf.at[i + 1]], buf1, sem1)
                pltpu.make_async_copy(gsrc.at[pl.ds(0, BE)], buf0, sem0).wait()
                pltpu.sync_copy(buf0, acc.at[rowbuf.at[i]], add=True)

                @pl.when(i + 2 < 2 * carry)
                def _():
                    pltpu.async_copy(gsrc.at[colbuf.at[i + 2]], buf0, sem0)

                pltpu.make_async_copy(gsrc.at[pl.ds(0, BE)], buf1, sem1).wait()
                pltpu.sync_copy(buf1, acc.at[rowbuf.at[i + 1]], add=True)
                return carry

            lax.fori_loop(0, npairs, body, npairs)
        plsc.subcore_barrier()
        pltpu.sync_copy(acc.at[pl.ds(s * RPT, RPT)],
                        u_hbm.at[pl.ds(oct_ * OROWS * 4 + s * RPT, RPT)])
        plsc.subcore_barrier()


@functools.cache
def _spmm_kernel():
    return pl.kernel(
        _spmm_body,
        mesh=plsc.VectorSubcoreMesh(core_axis_name="c", subcore_axis_name="s"),
        out_type=jax.ShapeDtypeStruct((NPA * 4, 128), jnp.float32),
        scratch_types=[
            pltpu.VMEM((NBC, BE), jnp.int32),
            pltpu.VMEM((NBC, BE), jnp.int32),
            pltpu.VMEM((16,), jnp.int32),
            pltpu.VMEM((BE, 128), jnp.float32),
            pltpu.VMEM((BE, 128), jnp.float32),
            pltpu.VMEM_SHARED((ACC4, 128), jnp.float32),
            pltpu.SemaphoreType.DMA,
            pltpu.SemaphoreType.DMA,
        ],
    )


def _spmm_call(*args):
    return _spmm_kernel()(*args)


def _mlp_body(x_ref, w1_ref, b1_ref, w2_ref, b2_ref, o_ref):
    x = x_ref[...]
    h = _gelu(jnp.dot(x, w1_ref[...], preferred_element_type=jnp.float32) + b1_ref[...])
    xm = jnp.dot(h, w2_ref[...], preferred_element_type=jnp.float32) + b2_ref[...]
    xs = xm + x
    nrm = jnp.sqrt(jnp.sum(xs * xs, axis=1, keepdims=True))
    xn = xs / jnp.maximum(nrm, 1e-12)
    o_ref[...] = xn + xm


def _mlp_norm(x, w1, b1, w2, b2):
    m = x.shape[0]
    return pl.pallas_call(
        _mlp_body,
        grid=(m // ROW_BLK,),
        in_specs=[
            pl.BlockSpec((ROW_BLK, H), lambda i: (i, 0)),
            pl.BlockSpec((H, H), lambda i: (0, 0)),
            pl.BlockSpec((1, H), lambda i: (0, 0)),
            pl.BlockSpec((H, H), lambda i: (0, 0)),
            pl.BlockSpec((1, H), lambda i: (0, 0)),
        ],
        out_specs=pl.BlockSpec((ROW_BLK, H), lambda i: (i, 0)),
        out_shape=jax.ShapeDtypeStruct((m, H), jnp.float32),
    )(x, w1, b1.reshape(1, H), w2, b2.reshape(1, H))


CPAD = 128


def _head_body(x_ref, w1_ref, b1_ref, w2_ref, b2_ref, w3_ref, b3_ref,
               w4_ref, b4_ref, o_ref):
    h = _gelu(jnp.dot(x_ref[...], w1_ref[...], preferred_element_type=jnp.float32) + b1_ref[...])
    h = _gelu(jnp.dot(h, w2_ref[...], preferred_element_type=jnp.float32) + b2_ref[...])
    h = _gelu(jnp.dot(h, w3_ref[...], preferred_element_type=jnp.float32) + b3_ref[...])
    logits = jnp.dot(h, w4_ref[...], preferred_element_type=jnp.float32) + b4_ref[...]
    mx = jnp.max(logits, axis=1, keepdims=True)
    z = logits - mx
    lse = jnp.log(jnp.sum(jnp.exp(z), axis=1, keepdims=True))
    o_ref[...] = z - lse


def _head(x, w1, b1, w2, b2, w3, b3, w4, b4):
    m = x.shape[0]
    w4p = jnp.zeros((H, CPAD), jnp.float32).at[:, :C].set(w4)
    b4p = jnp.full((1, CPAD), -1e30, jnp.float32).at[0, :C].set(b4)
    out = pl.pallas_call(
        _head_body,
        grid=(m // ROW_BLK,),
        in_specs=[
            pl.BlockSpec((ROW_BLK, H), lambda i: (i, 0)),
            pl.BlockSpec((H, H), lambda i: (0, 0)),
            pl.BlockSpec((1, H), lambda i: (0, 0)),
            pl.BlockSpec((H, H), lambda i: (0, 0)),
            pl.BlockSpec((1, H), lambda i: (0, 0)),
            pl.BlockSpec((H, H), lambda i: (0, 0)),
            pl.BlockSpec((1, H), lambda i: (0, 0)),
            pl.BlockSpec((H, CPAD), lambda i: (0, 0)),
            pl.BlockSpec((1, CPAD), lambda i: (0, 0)),
        ],
        out_specs=pl.BlockSpec((ROW_BLK, CPAD), lambda i: (i, 0)),
        out_shape=jax.ShapeDtypeStruct((m, CPAD), jnp.float32),
    )(x, w1, b1.reshape(1, H), w2, b2.reshape(1, H), w3, b3.reshape(1, H),
      w4p, b4p)
    return out[:, :C]


def _edge_slabs(src, dst):
    quart = jnp.arange(4, dtype=jnp.int32)

    def part(col, row):
        oct_ = row // OROWS
        local = row - oct_ * OROWS
        q = jnp.zeros((E,), jnp.int32)
        for o in range(8):
            m = (oct_ == o).astype(jnp.int32)
            q = q + m * (jnp.cumsum(m) - 1)
        pos = oct_ * (16 * CAPT) + (q % 16) * CAPT + q // 16
        pos4 = (pos[:, None] * 4 + quart[None, :]).reshape(-1)
        col4 = (col[:, None] * 4 + quart[None, :]).reshape(-1)
        loc4 = (local[:, None] * 4 + quart[None, :]).reshape(-1)
        pad_c = jnp.tile(quart, 8 * 16 * CAPT)
        pad_r = 4 * OROWS + pad_c
        cols_arr = pad_c.at[pos4].set(col4)
        rows_arr = pad_r.at[pos4].set(loc4)
        cnt = jnp.zeros((8,), jnp.int32).at[oct_].add(1)
        t = jnp.arange(16, dtype=jnp.int32)
        per_tile = jnp.maximum(cnt[:, None] - t[None, :] + 15, 0) // 16
        nbatch = (per_tile + KB - 1) // KB
        npair = jnp.maximum((nbatch + 1) // 2, 1)
        return (cols_arr.reshape(8, 16, NBC, BE).astype(jnp.int32),
                rows_arr.reshape(8, 16, NBC, BE).astype(jnp.int32),
                npair.astype(jnp.int32))

    colF, rowsF, npF = part(src, dst)
    colB, rowsB, npB = part(dst, src)
    nbp = jnp.zeros((2, 16, 16), jnp.int32)
    npF2 = npF.reshape(2, 4, 16).transpose(0, 2, 1)
    npB2 = npB.reshape(2, 4, 16).transpose(0, 2, 1)
    nbp = nbp.at[:, :, 0:4].set(npF2).at[:, :, 4:8].set(npB2)
    return colF, colB, rowsF, rowsB, nbp


def _conv(x, w, b, residual, a2d, cols_rows, zerosrow):
    colsF, colsB, rowsF, rowsB, nbp = cols_rows
    h = _proj(x, w, b)
    gp, gm = _ginit(h, a2d)
    out = h
    for t in range(1, T + 1):
        u4 = _spmm_call(gp.reshape(NP2 * 4, 128), gm.reshape(NP2 * 4, 128),
                        colsF, colsB, rowsF, rowsB, nbp, zerosrow)
        u = u4.reshape(NPA, H)
        if t < T:
            out, gp, gm = _term_update(u, a2d, out, t)
        else:
            out = _term_final(u, a2d, out, x if residual else None, t)
    return out


def kernel(x_in, edge_index, convW0, convb0, convW, convb, mlpW1, mlpb1,
           mlpW2, mlpb2, finW1, finb1, finW2, finb2, finW3, finb3,
           finW4, finb4):
    src = edge_index[0]
    dst = edge_index[1]
    deg = jnp.zeros((N,), jnp.float32).at[src].add(1.0).at[dst].add(1.0)
    a2d = (1.0 / jnp.sqrt(deg + 1.0)).reshape(N, 1)
    cols_rows = _edge_slabs(src, dst)
    zerosrow = jnp.zeros((64, 128), jnp.float32)

    x = x_in
    for i in range(3):
        if i == 0:
            x = _conv(x, convW0, convb0, False, a2d, cols_rows, zerosrow)
        else:
            x = _conv(x, convW[i - 1], convb[i - 1], True, a2d, cols_rows,
                      zerosrow)
        x = _mlp_norm(x, mlpW1[i], mlpb1[i], mlpW2[i], mlpb2[i])
    return _head(x, finW1, finb1, finW2, finb2, finW3, finb3, finW4, finb4)
